# baseline (speedup 1.0000x reference)
import functools

import jax
import jax.numpy as jnp
from jax import lax
from jax.experimental import pallas as pl
from jax.experimental.pallas import tpu as pltpu
from jax.experimental.pallas import tpu_sc as plsc

B = 1024
L = 200
D = 64
NC = 2
NS = 16
NW = NC * NS
ROWS_PER_W = B // NW
RB = 4
NBLK = ROWS_PER_W // RB
BLK = RB * L
LANES = 16


def kernel(input, token_table, pos_table):
    idx_flat = input.reshape(B * L)
    mesh = plsc.VectorSubcoreMesh(core_axis_name="c", subcore_axis_name="s")

    @functools.partial(
        pl.kernel,
        out_type=jax.ShapeDtypeStruct((B * L, D), jnp.float32),
        mesh=mesh,
        compiler_params=pltpu.CompilerParams(use_tc_tiling_on_sc=False),
        scratch_types=[
            pltpu.VMEM((L * ROWS_PER_W,), jnp.int32),
            pltpu.VMEM((L, D), jnp.float32),
            pltpu.VMEM((BLK, D), jnp.float32),
            pltpu.VMEM((BLK, D), jnp.float32),
            pltpu.SemaphoreType.DMA,
            pltpu.SemaphoreType.DMA,
            pltpu.SemaphoreType.DMA,
            pltpu.SemaphoreType.DMA,
        ],
    )
    def emb_kernel(idx_hbm, tok_hbm, pos_hbm, out_hbm,
                   idx_v, pos_v, rows_a, rows_b, ga, gb, oa, ob):
        wid = lax.axis_index("s") * NC + lax.axis_index("c")
        base = wid * (L * ROWS_PER_W)
        pltpu.sync_copy(idx_hbm.at[pl.ds(base, L * ROWS_PER_W)], idx_v)
        pltpu.sync_copy(pos_hbm, pos_v)

        bufs = (rows_a, rows_b)
        gsems = (ga, gb)
        osems = (oa, ob)

        def issue_gather(blk, buf, gsem):
            pltpu.async_copy(tok_hbm.at[idx_v.at[pl.ds(blk * BLK, BLK)]],
                             buf, gsem)

        def wait_gather(buf, gsem):
            pltpu.make_async_copy(tok_hbm.at[idx_v.at[pl.ds(0, BLK)]],
                                  buf, gsem).wait()

        def wait_out(buf, osem):
            pltpu.make_async_copy(buf, out_hbm.at[pl.ds(base, BLK)],
                                  osem).wait()

        def add_pos(buf):
            for q in range(RB):
                @pl.loop(0, L, unroll=4)
                def _add_row(i):
                    for j in range(0, D, LANES):
                        src = (pl.ds(i, 1), pl.ds(j, LANES))
                        dst = (pl.ds(q * L + i, 1), pl.ds(j, LANES))
                        buf.at[*dst][...] = (
                            buf.at[*dst][...] + pos_v.at[*src][...])

        issue_gather(0, rows_a, ga)

        @pl.loop(0, NBLK, step=2)
        def _blk(b0):
            for t in range(2):
                blk = b0 + t
                buf, gsem, osem = bufs[t], gsems[t], osems[t]
                nbuf, ngsem, nosem = bufs[1 - t], gsems[1 - t], osems[1 - t]

                @pl.when(blk >= 1)
                def _():
                    wait_out(nbuf, nosem)

                @pl.when(blk + 1 < NBLK)
                def _():
                    issue_gather(blk + 1, nbuf, ngsem)

                wait_gather(buf, gsem)
                add_pos(buf)
                pltpu.async_copy(buf, out_hbm.at[pl.ds(base + blk * BLK, BLK)],
                                 osem)

        wait_out(rows_b, ob)

    out = emb_kernel(idx_flat, token_table, pos_table)
    return out.reshape(B, L, D)


# speedup vs baseline: 1.1025x; 1.1025x over previous
import functools

import jax
import jax.numpy as jnp
from jax import lax
from jax.experimental import pallas as pl
from jax.experimental.pallas import tpu as pltpu
from jax.experimental.pallas import tpu_sc as plsc

B = 1024
L = 200
D = 64
NC = 2
NS = 16
NW = NC * NS
ROWS_PER_W = B // NW
RB = 4
NBLK = ROWS_PER_W // RB
BLK = RB * L
LANES = 16


def kernel(input, token_table, pos_table):
    idx_flat = input.reshape(B * L)
    mesh = plsc.VectorSubcoreMesh(core_axis_name="c", subcore_axis_name="s")

    @functools.partial(
        pl.kernel,
        out_type=jax.ShapeDtypeStruct((B * L, 2 * D), jnp.float32),
        mesh=mesh,
        compiler_params=pltpu.CompilerParams(use_tc_tiling_on_sc=False),
        scratch_types=[
            pltpu.VMEM((L * ROWS_PER_W,), jnp.int32),
            pltpu.VMEM((L, D), jnp.float32),
            pltpu.VMEM((BLK, D), jnp.float32),
            pltpu.VMEM((BLK, D), jnp.float32),
            pltpu.SemaphoreType.DMA,
            pltpu.SemaphoreType.DMA,
            pltpu.SemaphoreType.DMA,
            pltpu.SemaphoreType.DMA,
        ],
    )
    def emb_kernel(idx_hbm, tok_hbm, pos_hbm, out_hbm,
                   idx_v, pos_v, rows_a, rows_b, ga, gb, oa, ob):
        wid = lax.axis_index("s") * NC + lax.axis_index("c")
        base = wid * (L * ROWS_PER_W)
        pltpu.sync_copy(idx_hbm.at[pl.ds(base, L * ROWS_PER_W)], idx_v)
        pltpu.sync_copy(pos_hbm, pos_v)

        bufs = (rows_a, rows_b)
        gsems = (ga, gb)
        osems = (oa, ob)

        def issue_gather(blk, buf, gsem):
            pltpu.async_copy(tok_hbm.at[idx_v.at[pl.ds(blk * BLK, BLK)]],
                             buf, gsem)

        def wait_gather(buf, gsem):
            pltpu.make_async_copy(tok_hbm.at[idx_v.at[pl.ds(0, BLK)]],
                                  buf, gsem).wait()

        def wait_out(buf, osem):
            pltpu.make_async_copy(
                buf, out_hbm.at[pl.ds(base, BLK), pl.ds(0, D)], osem).wait()

        def add_pos(buf):
            for q in range(RB):
                @pl.loop(0, L, unroll=4)
                def _add_row(i):
                    for j in range(0, D, LANES):
                        src = (pl.ds(i, 1), pl.ds(j, LANES))
                        dst = (pl.ds(q * L + i, 1), pl.ds(j, LANES))
                        buf.at[*dst][...] = (
                            buf.at[*dst][...] + pos_v.at[*src][...])

        issue_gather(0, rows_a, ga)

        @pl.loop(0, NBLK, step=2)
        def _blk(b0):
            for t in range(2):
                blk = b0 + t
                buf, gsem, osem = bufs[t], gsems[t], osems[t]
                nbuf, ngsem, nosem = bufs[1 - t], gsems[1 - t], osems[1 - t]

                @pl.when(blk >= 1)
                def _():
                    wait_out(nbuf, nosem)

                @pl.when(blk + 1 < NBLK)
                def _():
                    issue_gather(blk + 1, nbuf, ngsem)

                wait_gather(buf, gsem)
                add_pos(buf)
                pltpu.async_copy(
                    buf,
                    out_hbm.at[pl.ds(base + blk * BLK, BLK), pl.ds(0, D)],
                    osem)

        wait_out(rows_b, ob)

    out = emb_kernel(idx_flat, token_table, pos_table)
    return out.reshape(B, L, 2 * D)[:, :, :D]


# speedup vs baseline: 1.1110x; 1.0077x over previous
import functools

import jax
import jax.numpy as jnp
from jax import lax
from jax.experimental import pallas as pl
from jax.experimental.pallas import tpu as pltpu
from jax.experimental.pallas import tpu_sc as plsc

B = 1024
L = 200
D = 64
NC = 2
NS = 16
NW = NC * NS
ROWS_PER_W = B // NW
RB = 2
NBLK = ROWS_PER_W // RB
BLK = RB * L
LANES = 16


def kernel(input, token_table, pos_table):
    idx_flat = input.reshape(B * L)
    tt_pad = jnp.pad(token_table, ((0, 0), (0, D)))
    mesh = plsc.VectorSubcoreMesh(core_axis_name="c", subcore_axis_name="s")

    @functools.partial(
        pl.kernel,
        out_type=jax.ShapeDtypeStruct((B * L, 2 * D), jnp.float32),
        mesh=mesh,
        compiler_params=pltpu.CompilerParams(use_tc_tiling_on_sc=False),
        scratch_types=[
            pltpu.VMEM((L * ROWS_PER_W,), jnp.int32),
            pltpu.VMEM((L, D), jnp.float32),
            pltpu.VMEM((BLK, 2 * D), jnp.float32),
            pltpu.VMEM((BLK, 2 * D), jnp.float32),
            pltpu.SemaphoreType.DMA,
            pltpu.SemaphoreType.DMA,
            pltpu.SemaphoreType.DMA,
            pltpu.SemaphoreType.DMA,
        ],
    )
    def emb_kernel(idx_hbm, tok_hbm, pos_hbm, out_hbm,
                   idx_v, pos_v, rows_a, rows_b, ga, gb, oa, ob):
        wid = lax.axis_index("s") * NC + lax.axis_index("c")
        base = wid * (L * ROWS_PER_W)
        pltpu.sync_copy(idx_hbm.at[pl.ds(base, L * ROWS_PER_W)], idx_v)
        pltpu.sync_copy(pos_hbm, pos_v)

        bufs = (rows_a, rows_b)
        gsems = (ga, gb)
        osems = (oa, ob)

        def issue_gather(blk, buf, gsem):
            pltpu.async_copy(tok_hbm.at[idx_v.at[pl.ds(blk * BLK, BLK)]],
                             buf, gsem)

        def wait_gather(buf, gsem):
            pltpu.make_async_copy(tok_hbm.at[idx_v.at[pl.ds(0, BLK)]],
                                  buf, gsem).wait()

        def wait_out(buf, osem):
            pltpu.make_async_copy(buf, out_hbm.at[pl.ds(base, BLK)],
                                  osem).wait()

        def add_pos(buf):
            for q in range(RB):
                @pl.loop(0, L, unroll=4)
                def _add_row(i):
                    for j in range(0, D, LANES):
                        src = (pl.ds(i, 1), pl.ds(j, LANES))
                        dst = (pl.ds(q * L + i, 1), pl.ds(j, LANES))
                        buf.at[*dst][...] = (
                            buf.at[*dst][...] + pos_v.at[*src][...])

        issue_gather(0, rows_a, ga)

        @pl.loop(0, NBLK, step=2)
        def _blk(b0):
            for t in range(2):
                blk = b0 + t
                buf, gsem, osem = bufs[t], gsems[t], osems[t]
                nbuf, ngsem, nosem = bufs[1 - t], gsems[1 - t], osems[1 - t]

                @pl.when(blk >= 1)
                def _():
                    wait_out(nbuf, nosem)

                @pl.when(blk + 1 < NBLK)
                def _():
                    issue_gather(blk + 1, nbuf, ngsem)

                wait_gather(buf, gsem)
                add_pos(buf)
                pltpu.async_copy(
                    buf, out_hbm.at[pl.ds(base + blk * BLK, BLK)], osem)

        wait_out(rows_b, ob)

    out = emb_kernel(idx_flat, tt_pad, pos_table)
    return out.reshape(B, L, 2 * D)[:, :, :D]


# speedup vs baseline: 1.2393x; 1.1155x over previous
import functools

import jax
import jax.numpy as jnp
from jax import lax
from jax.experimental import pallas as pl
from jax.experimental.pallas import tpu as pltpu
from jax.experimental.pallas import tpu_sc as plsc

B = 1024
L = 200
D = 64
NC = 2
NS = 16
NW = NC * NS
ROWS_PER_W = B // NW
RB = 2
NBLK = ROWS_PER_W // RB
BLK = RB * L
LANES = 16


def kernel(input, token_table, pos_table):
    idx_flat = input.reshape(B * L)
    tt_pad = jnp.pad(token_table, ((0, 0), (0, D)))
    mesh = plsc.VectorSubcoreMesh(core_axis_name="c", subcore_axis_name="s")

    @functools.partial(
        pl.kernel,
        out_type=jax.ShapeDtypeStruct((B * L, 2 * D), jnp.float32),
        mesh=mesh,
        compiler_params=pltpu.CompilerParams(use_tc_tiling_on_sc=False),
        scratch_types=[
            pltpu.VMEM((L * ROWS_PER_W,), jnp.int32),
            pltpu.VMEM((L, D), jnp.float32),
            pltpu.VMEM((BLK, 2 * D), jnp.float32),
            pltpu.VMEM((BLK, 2 * D), jnp.float32),
            pltpu.SemaphoreType.DMA,
            pltpu.SemaphoreType.DMA,
            pltpu.SemaphoreType.DMA,
            pltpu.SemaphoreType.DMA,
        ],
    )
    def emb_kernel(idx_hbm, tok_hbm, pos_hbm, out_hbm,
                   idx_v, pos_v, rows_a, rows_b, ga, gb, oa, ob):
        wid = lax.axis_index("s") * NC + lax.axis_index("c")
        base = wid * (L * ROWS_PER_W)
        pltpu.sync_copy(idx_hbm.at[pl.ds(base, L * ROWS_PER_W)], idx_v)
        pltpu.sync_copy(pos_hbm, pos_v)

        bufs = (rows_a, rows_b)
        gsems = (ga, gb)
        osems = (oa, ob)

        def issue_gather(blk, buf, gsem):
            pltpu.async_copy(tok_hbm.at[idx_v.at[pl.ds(blk * BLK, BLK)]],
                             buf, gsem)

        def wait_gather(buf, gsem):
            pltpu.make_async_copy(tok_hbm.at[idx_v.at[pl.ds(0, BLK)]],
                                  buf, gsem).wait()

        def wait_out(buf, osem):
            pltpu.make_async_copy(buf, out_hbm.at[pl.ds(base, BLK)],
                                  osem).wait()

        def add_pos(buf):
            for q in range(RB):
                @plsc.parallel_loop(0, L, unroll=4)
                def _add_row(i):
                    for j in range(0, D, LANES):
                        src = (pl.ds(i, 1), pl.ds(j, LANES))
                        dst = (pl.ds(q * L + i, 1), pl.ds(j, LANES))
                        buf.at[*dst][...] = (
                            buf.at[*dst][...] + pos_v.at[*src][...])

        issue_gather(0, rows_a, ga)

        @pl.loop(0, NBLK, step=2)
        def _blk(b0):
            for t in range(2):
                blk = b0 + t
                buf, gsem, osem = bufs[t], gsems[t], osems[t]
                nbuf, ngsem, nosem = bufs[1 - t], gsems[1 - t], osems[1 - t]

                @pl.when(blk >= 1)
                def _():
                    wait_out(nbuf, nosem)

                @pl.when(blk + 1 < NBLK)
                def _():
                    issue_gather(blk + 1, nbuf, ngsem)

                wait_gather(buf, gsem)
                add_pos(buf)
                pltpu.async_copy(
                    buf, out_hbm.at[pl.ds(base + blk * BLK, BLK)], osem)

        wait_out(rows_b, ob)

    out = emb_kernel(idx_flat, tt_pad, pos_table)
    return out.reshape(B, L, 2 * D)[:, :, :D]
